# baseline (speedup 1.0000x reference)
import functools

import jax
import jax.numpy as jnp
from jax import lax
from jax.experimental import pallas as pl
from jax.experimental.pallas import tpu as pltpu
from jax.experimental.pallas import tpu_sc as plsc

N = 10000
E = 320000
D = 128

NC = 2
NS = 16
NW = NC * NS
EPW = E // NW
CH = 80
NCHUNK = EPW // CH
CHC = 200
NCHUNKC = EPW // CHC
RPAD = 632
NPAD = NS * RPAD
CNTW = 8


@functools.lru_cache(maxsize=None)
def _sc_segsum_kernel():
  def body(y_hbm, src_hbm, dst_hbm, zrow_hbm, out_hbm,
           sidx0, sidx1, didx0, didx1, rows0, rows1, acc_sh, sem0, sem1):
    c = lax.axis_index("c")
    s = lax.axis_index("s")
    wid = c * NS + s

    pltpu.sync_copy(zrow_hbm, acc_sh.at[pl.ds(s * RPAD, RPAD)])
    plsc.subcore_barrier()

    def issue(j, sbuf, dbuf, rbuf, sem):
      base = wid * EPW + j * CH
      pltpu.sync_copy(src_hbm.at[pl.ds(base, CH)], sbuf)
      pltpu.sync_copy(dst_hbm.at[pl.ds(base, CH)], dbuf)
      pltpu.async_copy(y_hbm.at[sbuf], rbuf, sem)

    def drain_scatter(sbuf, dbuf, rbuf, sem):
      pltpu.make_async_copy(y_hbm.at[sbuf], rbuf, sem).wait()
      pltpu.sync_copy(rbuf, acc_sh.at[dbuf], add=True)

    issue(0, sidx0, didx0, rows0, sem0)

    def step(i, carry):
      issue(2 * i + 1, sidx1, didx1, rows1, sem1)
      drain_scatter(sidx0, didx0, rows0, sem0)
      issue(2 * i + 2, sidx0, didx0, rows0, sem0)
      drain_scatter(sidx1, didx1, rows1, sem1)
      return carry

    lax.fori_loop(0, (NCHUNK - 1) // 2, step, 0)
    drain_scatter(sidx0, didx0, rows0, sem0)
    plsc.subcore_barrier()

    pltpu.sync_copy(acc_sh.at[pl.ds(s * RPAD, RPAD)],
                    out_hbm.at[c, pl.ds(s * RPAD, RPAD)])

  mesh = plsc.VectorSubcoreMesh(core_axis_name="c", subcore_axis_name="s")
  return pl.kernel(
      body,
      out_type=[jax.ShapeDtypeStruct((NC, NPAD, D), jnp.float32)],
      mesh=mesh,
      scratch_types=[
          pltpu.VMEM((CH,), jnp.int32),
          pltpu.VMEM((CH,), jnp.int32),
          pltpu.VMEM((CH,), jnp.int32),
          pltpu.VMEM((CH,), jnp.int32),
          pltpu.VMEM((CH, D), jnp.float32),
          pltpu.VMEM((CH, D), jnp.float32),
          pltpu.VMEM_SHARED((NPAD, D), jnp.float32),
          pltpu.SemaphoreType.DMA,
          pltpu.SemaphoreType.DMA,
      ])


@functools.lru_cache(maxsize=None)
def _sc_count_kernel():
  def body(dst_hbm, zcnt_hbm, ones_hbm, cnt_out_hbm,
           didx, ones_v, cnt_sh):
    c = lax.axis_index("c")
    s = lax.axis_index("s")
    wid = c * NS + s

    pltpu.sync_copy(zcnt_hbm, cnt_sh.at[pl.ds(s * RPAD, RPAD)])
    pltpu.sync_copy(ones_hbm, ones_v)
    plsc.subcore_barrier()

    def step(j, carry):
      base = wid * EPW + j * CHC
      pltpu.sync_copy(dst_hbm.at[pl.ds(base, CHC)], didx)
      pltpu.sync_copy(ones_v, cnt_sh.at[didx], add=True)
      return carry

    lax.fori_loop(0, NCHUNKC, step, 0)
    plsc.subcore_barrier()

    pltpu.sync_copy(cnt_sh.at[pl.ds(s * RPAD, RPAD)],
                    cnt_out_hbm.at[c, pl.ds(s * RPAD, RPAD)])

  mesh = plsc.VectorSubcoreMesh(core_axis_name="c", subcore_axis_name="s")
  return pl.kernel(
      body,
      out_type=[jax.ShapeDtypeStruct((NC, NPAD, CNTW), jnp.float32)],
      mesh=mesh,
      scratch_types=[
          pltpu.VMEM((CHC,), jnp.int32),
          pltpu.VMEM((CHC, CNTW), jnp.float32),
          pltpu.VMEM_SHARED((NPAD, CNTW), jnp.float32),
      ],
      compiler_params=pltpu.CompilerParams(use_tc_tiling_on_sc=False))


_RB = 2000
_GRID = N // _RB


def _tc_pre_body(x_ref, wl_ref, wr_ref, b_ref, y_ref, r_ref):
  x = x_ref[...]
  y_ref[...] = jnp.dot(x, wl_ref[...], preferred_element_type=jnp.float32)
  r_ref[...] = (jnp.dot(x, wr_ref[...], preferred_element_type=jnp.float32)
                + b_ref[...])


def _tc_mid_body(z_ref, c_ref, r_ref, wl_ref, wr_ref,
                 b_ref, y_ref, r2_ref):
  cnt = c_ref[0, :, 0:1] + c_ref[1, :, 0:1]
  mean = (z_ref[0] + z_ref[1]) / jnp.maximum(cnt, 1.0)
  h = jnp.maximum(mean + r_ref[...], 0.0)
  y_ref[...] = jnp.dot(h, wl_ref[...], preferred_element_type=jnp.float32)
  r2_ref[...] = (jnp.dot(h, wr_ref[...], preferred_element_type=jnp.float32)
                 + b_ref[...])


def _tc_post_body(z_ref, c_ref, r_ref, o_ref):
  cnt = c_ref[0, :, 0:1] + c_ref[1, :, 0:1]
  o_ref[...] = (z_ref[0] + z_ref[1]) / jnp.maximum(cnt, 1.0) + r_ref[...]


def _row_spec(width):
  return pl.BlockSpec((_RB, width), lambda i: (i, 0))


def _pad_spec(width):
  return pl.BlockSpec((NC, _RB, width), lambda i: (0, i, 0))


def _full_spec(rows, cols):
  return pl.BlockSpec((rows, cols), lambda i: (0, 0))


_dense_shape = jax.ShapeDtypeStruct((N, D), jnp.float32)

_tc_pre = pl.pallas_call(
    _tc_pre_body,
    grid=(_GRID,),
    in_specs=[_row_spec(D), _full_spec(D, D), _full_spec(D, D),
              _full_spec(1, D)],
    out_specs=[_row_spec(D), _row_spec(D)],
    out_shape=[_dense_shape, _dense_shape],
)

_tc_mid = pl.pallas_call(
    _tc_mid_body,
    grid=(_GRID,),
    in_specs=[_pad_spec(D), _pad_spec(CNTW),
              _row_spec(D), _full_spec(D, D), _full_spec(D, D),
              _full_spec(1, D)],
    out_specs=[_row_spec(D), _row_spec(D)],
    out_shape=[_dense_shape, _dense_shape],
)

_tc_post = pl.pallas_call(
    _tc_post_body,
    grid=(_GRID,),
    in_specs=[_pad_spec(D), _pad_spec(CNTW), _row_spec(D)],
    out_specs=_row_spec(D),
    out_shape=_dense_shape,
)


@jax.jit
def kernel(x, edge_index, W1l, W1r, b1, W2l, W2r, b2):
  src = edge_index[0]
  dst = edge_index[1]
  b1r = b1.reshape(1, D)
  b2r = b2.reshape(1, D)
  zrow = jnp.zeros((RPAD, D), jnp.float32)
  zcnt = jnp.zeros((RPAD, CNTW), jnp.float32)
  ones = jnp.ones((CHC, CNTW), jnp.float32)

  (cnt,) = _sc_count_kernel()(dst, zcnt, ones)
  y1, r1 = _tc_pre(x, W1l, W1r, b1r)
  (z1,) = _sc_segsum_kernel()(y1, src, dst, zrow)
  y2, r2 = _tc_mid(z1, cnt, r1, W2l, W2r, b2r)
  (z2,) = _sc_segsum_kernel()(y2, src, dst, zrow)
  out = _tc_post(z2, cnt, r2)
  return out


# speedup vs baseline: 1.4707x; 1.4707x over previous
import functools

import jax
import jax.numpy as jnp
from jax import lax
from jax.experimental import pallas as pl
from jax.experimental.pallas import tpu as pltpu
from jax.experimental.pallas import tpu_sc as plsc

N = 10000
E = 320000
D = 128

NC = 2
NS = 16
NW = NC * NS
EPW = E // NW
CH = 80
NCHUNK = EPW // CH
CHC = 200
NCHUNKC = EPW // CHC
RPAD = 632
NPAD = NS * RPAD
CNTW = 8


@functools.lru_cache(maxsize=None)
def _sc_segsum_kernel():
  def body(y_hbm, src_hbm, dst_hbm, zrow_hbm, out_hbm,
           sidx_all, didx_all, rows0, rows1, acc_sh, sem0, sem1):
    c = lax.axis_index("c")
    s = lax.axis_index("s")
    wid = c * NS + s

    pltpu.sync_copy(zrow_hbm, acc_sh.at[pl.ds(s * RPAD, RPAD)])
    pltpu.sync_copy(src_hbm.at[wid], sidx_all)
    pltpu.sync_copy(dst_hbm.at[wid], didx_all)
    plsc.subcore_barrier()

    def issue(j, rbuf, sem):
      pltpu.async_copy(y_hbm.at[sidx_all.at[pl.ds(j * CH, CH)]], rbuf, sem)

    def drain_scatter(j, rbuf, sem):
      pltpu.make_async_copy(
          y_hbm.at[sidx_all.at[pl.ds(j * CH, CH)]], rbuf, sem).wait()
      pltpu.sync_copy(rbuf, acc_sh.at[didx_all.at[j]], add=True)

    issue(0, rows0, sem0)

    def step(i, carry):
      issue(2 * i + 1, rows1, sem1)
      drain_scatter(2 * i, rows0, sem0)
      issue(2 * i + 2, rows0, sem0)
      drain_scatter(2 * i + 1, rows1, sem1)
      return carry

    lax.fori_loop(0, (NCHUNK - 1) // 2, step, 0)
    drain_scatter(NCHUNK - 1, rows0, sem0)
    plsc.subcore_barrier()

    pltpu.sync_copy(acc_sh.at[pl.ds(s * RPAD, RPAD)],
                    out_hbm.at[c, pl.ds(s * RPAD, RPAD)])

  mesh = plsc.VectorSubcoreMesh(core_axis_name="c", subcore_axis_name="s")
  return pl.kernel(
      body,
      out_type=[jax.ShapeDtypeStruct((NC, NPAD, D), jnp.float32)],
      mesh=mesh,
      scratch_types=[
          pltpu.VMEM((EPW,), jnp.int32),
          pltpu.VMEM((NCHUNK, CH), jnp.int32),
          pltpu.VMEM((CH, D), jnp.float32),
          pltpu.VMEM((CH, D), jnp.float32),
          pltpu.VMEM_SHARED((NPAD, D), jnp.float32),
          pltpu.SemaphoreType.DMA,
          pltpu.SemaphoreType.DMA,
      ])


@functools.lru_cache(maxsize=None)
def _sc_count_kernel():
  def body(dst_hbm, zcnt_hbm, ones_hbm, cnt_out_hbm,
           didx_all, ones_v, cnt_sh):
    c = lax.axis_index("c")
    s = lax.axis_index("s")
    wid = c * NS + s

    pltpu.sync_copy(zcnt_hbm, cnt_sh.at[pl.ds(s * RPAD, RPAD)])
    pltpu.sync_copy(ones_hbm, ones_v)
    pltpu.sync_copy(dst_hbm.at[wid], didx_all)
    plsc.subcore_barrier()

    def step(j, carry):
      pltpu.sync_copy(ones_v, cnt_sh.at[didx_all.at[j]], add=True)
      return carry

    lax.fori_loop(0, NCHUNKC, step, 0)
    plsc.subcore_barrier()

    pltpu.sync_copy(cnt_sh.at[pl.ds(s * RPAD, RPAD)],
                    cnt_out_hbm.at[c, pl.ds(s * RPAD, RPAD)])

  mesh = plsc.VectorSubcoreMesh(core_axis_name="c", subcore_axis_name="s")
  return pl.kernel(
      body,
      out_type=[jax.ShapeDtypeStruct((NC, NPAD, CNTW), jnp.float32)],
      mesh=mesh,
      scratch_types=[
          pltpu.VMEM((NCHUNKC, CHC), jnp.int32),
          pltpu.VMEM((CHC, CNTW), jnp.float32),
          pltpu.VMEM_SHARED((NPAD, CNTW), jnp.float32),
      ],
      compiler_params=pltpu.CompilerParams(use_tc_tiling_on_sc=False))


_RB = 2000
_GRID = N // _RB


def _tc_pre_body(x_ref, wl_ref, wr_ref, b_ref, y_ref, r_ref):
  x = x_ref[...]
  y_ref[...] = jnp.dot(x, wl_ref[...], preferred_element_type=jnp.float32)
  r_ref[...] = (jnp.dot(x, wr_ref[...], preferred_element_type=jnp.float32)
                + b_ref[...])


def _tc_mid_body(z_ref, c_ref, r_ref, wl_ref, wr_ref,
                 b_ref, y_ref, r2_ref):
  cnt = c_ref[0, :, 0:1] + c_ref[1, :, 0:1]
  mean = (z_ref[0] + z_ref[1]) / jnp.maximum(cnt, 1.0)
  h = jnp.maximum(mean + r_ref[...], 0.0)
  y_ref[...] = jnp.dot(h, wl_ref[...], preferred_element_type=jnp.float32)
  r2_ref[...] = (jnp.dot(h, wr_ref[...], preferred_element_type=jnp.float32)
                 + b_ref[...])


def _tc_post_body(z_ref, c_ref, r_ref, o_ref):
  cnt = c_ref[0, :, 0:1] + c_ref[1, :, 0:1]
  o_ref[...] = (z_ref[0] + z_ref[1]) / jnp.maximum(cnt, 1.0) + r_ref[...]


def _row_spec(width):
  return pl.BlockSpec((_RB, width), lambda i: (i, 0))


def _pad_spec(width):
  return pl.BlockSpec((NC, _RB, width), lambda i: (0, i, 0))


def _full_spec(rows, cols):
  return pl.BlockSpec((rows, cols), lambda i: (0, 0))


_dense_shape = jax.ShapeDtypeStruct((N, D), jnp.float32)

_tc_pre = pl.pallas_call(
    _tc_pre_body,
    grid=(_GRID,),
    in_specs=[_row_spec(D), _full_spec(D, D), _full_spec(D, D),
              _full_spec(1, D)],
    out_specs=[_row_spec(D), _row_spec(D)],
    out_shape=[_dense_shape, _dense_shape],
)

_tc_mid = pl.pallas_call(
    _tc_mid_body,
    grid=(_GRID,),
    in_specs=[_pad_spec(D), _pad_spec(CNTW),
              _row_spec(D), _full_spec(D, D), _full_spec(D, D),
              _full_spec(1, D)],
    out_specs=[_row_spec(D), _row_spec(D)],
    out_shape=[_dense_shape, _dense_shape],
)

_tc_post = pl.pallas_call(
    _tc_post_body,
    grid=(_GRID,),
    in_specs=[_pad_spec(D), _pad_spec(CNTW), _row_spec(D)],
    out_specs=_row_spec(D),
    out_shape=_dense_shape,
)


@jax.jit
def kernel(x, edge_index, W1l, W1r, b1, W2l, W2r, b2):
  src = edge_index[0].reshape(NW, EPW)
  dst = edge_index[1].reshape(NW, NCHUNK, CH)
  dstc = edge_index[1].reshape(NW, NCHUNKC, CHC)
  b1r = b1.reshape(1, D)
  b2r = b2.reshape(1, D)
  zrow = jnp.zeros((RPAD, D), jnp.float32)
  zcnt = jnp.zeros((RPAD, CNTW), jnp.float32)
  ones = jnp.ones((CHC, CNTW), jnp.float32)

  (cnt,) = _sc_count_kernel()(dstc, zcnt, ones)
  y1, r1 = _tc_pre(x, W1l, W1r, b1r)
  (z1,) = _sc_segsum_kernel()(y1, src, dst, zrow)
  y2, r2 = _tc_mid(z1, cnt, r1, W2l, W2r, b2r)
  (z2,) = _sc_segsum_kernel()(y2, src, dst, zrow)
  out = _tc_post(z2, cnt, r2)
  return out


# speedup vs baseline: 1.5504x; 1.0542x over previous
import functools

import jax
import jax.numpy as jnp
from jax import lax
from jax.experimental import pallas as pl
from jax.experimental.pallas import tpu as pltpu
from jax.experimental.pallas import tpu_sc as plsc

N = 10000
E = 320000
D = 128

NC = 2
NS = 16
NW = NC * NS
EPW = E // NW
CH = 104
NCHUNK = 97
EPP = NCHUNK * CH
CHC = 200
NCHUNKC = EPW // CHC
RPAD = 632
NPAD = NS * RPAD
CNTW = 8


@functools.lru_cache(maxsize=None)
def _sc_segsum_kernel():
  def body(y_hbm, src_hbm, dst_hbm, zrow_hbm, out_hbm,
           sidx_all, didx_all, rows0, rows1, acc_sh, gsem0, gsem1):
    c = lax.axis_index("c")
    s = lax.axis_index("s")
    wid = c * NS + s

    pltpu.sync_copy(zrow_hbm, acc_sh.at[pl.ds(s * RPAD, RPAD)])
    pltpu.sync_copy(src_hbm.at[wid], sidx_all)
    pltpu.sync_copy(dst_hbm.at[wid], didx_all)
    plsc.subcore_barrier()

    def issue(j, rbuf, sem):
      pltpu.async_copy(y_hbm.at[sidx_all.at[pl.ds(j * CH, CH)]], rbuf, sem)

    def drain_scatter(j, rbuf, sem):
      pltpu.make_async_copy(
          y_hbm.at[sidx_all.at[pl.ds(j * CH, CH)]], rbuf, sem).wait()
      pltpu.sync_copy(rbuf, acc_sh.at[didx_all.at[j]], add=True)

    issue(0, rows0, gsem0)

    def step(i, carry):
      issue(2 * i + 1, rows1, gsem1)
      drain_scatter(2 * i, rows0, gsem0)
      issue(2 * i + 2, rows0, gsem0)
      drain_scatter(2 * i + 1, rows1, gsem1)
      return carry

    lax.fori_loop(0, (NCHUNK - 1) // 2, step, 0)
    drain_scatter(NCHUNK - 1, rows0, gsem0)
    plsc.subcore_barrier()

    pltpu.sync_copy(acc_sh.at[pl.ds(s * RPAD, RPAD)],
                    out_hbm.at[c, pl.ds(s * RPAD, RPAD)])

  mesh = plsc.VectorSubcoreMesh(core_axis_name="c", subcore_axis_name="s")
  return pl.kernel(
      body,
      out_type=[jax.ShapeDtypeStruct((NC, NPAD, D), jnp.float32)],
      mesh=mesh,
      scratch_types=[
          pltpu.VMEM((EPP,), jnp.int32),
          pltpu.VMEM((NCHUNK, CH), jnp.int32),
          pltpu.VMEM((CH, D), jnp.float32),
          pltpu.VMEM((CH, D), jnp.float32),
          pltpu.VMEM_SHARED((NPAD, D), jnp.float32),
          pltpu.SemaphoreType.DMA,
          pltpu.SemaphoreType.DMA,
      ])


@functools.lru_cache(maxsize=None)
def _sc_count_kernel():
  def body(dst_hbm, zcnt_hbm, ones_hbm, cnt_out_hbm,
           didx_all, ones_v, cnt_sh):
    c = lax.axis_index("c")
    s = lax.axis_index("s")
    wid = c * NS + s

    pltpu.sync_copy(zcnt_hbm, cnt_sh.at[pl.ds(s * RPAD, RPAD)])
    pltpu.sync_copy(ones_hbm, ones_v)
    pltpu.sync_copy(dst_hbm.at[wid], didx_all)
    plsc.subcore_barrier()

    def step(j, carry):
      pltpu.sync_copy(ones_v, cnt_sh.at[didx_all.at[j]], add=True)
      return carry

    lax.fori_loop(0, NCHUNKC, step, 0)
    plsc.subcore_barrier()

    pltpu.sync_copy(cnt_sh.at[pl.ds(s * RPAD, RPAD)],
                    cnt_out_hbm.at[c, pl.ds(s * RPAD, RPAD)])

  mesh = plsc.VectorSubcoreMesh(core_axis_name="c", subcore_axis_name="s")
  return pl.kernel(
      body,
      out_type=[jax.ShapeDtypeStruct((NC, NPAD, CNTW), jnp.float32)],
      mesh=mesh,
      scratch_types=[
          pltpu.VMEM((NCHUNKC, CHC), jnp.int32),
          pltpu.VMEM((CHC, CNTW), jnp.float32),
          pltpu.VMEM_SHARED((NPAD, CNTW), jnp.float32),
      ],
      compiler_params=pltpu.CompilerParams(use_tc_tiling_on_sc=False))


_RB = 2000
_GRID = N // _RB


def _tc_pre_body(x_ref, wl_ref, wr_ref, b_ref, y_ref, r_ref):
  x = x_ref[...]
  y_ref[...] = jnp.dot(x, wl_ref[...], preferred_element_type=jnp.float32)
  r_ref[...] = (jnp.dot(x, wr_ref[...], preferred_element_type=jnp.float32)
                + b_ref[...])


def _tc_mid_body(z_ref, c_ref, r_ref, wl_ref, wr_ref,
                 b_ref, y_ref, r2_ref):
  cnt = c_ref[0, :, 0:1] + c_ref[1, :, 0:1]
  mean = (z_ref[0] + z_ref[1]) / jnp.maximum(cnt, 1.0)
  h = jnp.maximum(mean + r_ref[...], 0.0)
  y_ref[...] = jnp.dot(h, wl_ref[...], preferred_element_type=jnp.float32)
  r2_ref[...] = (jnp.dot(h, wr_ref[...], preferred_element_type=jnp.float32)
                 + b_ref[...])


def _tc_post_body(z_ref, c_ref, r_ref, o_ref):
  cnt = c_ref[0, :, 0:1] + c_ref[1, :, 0:1]
  o_ref[...] = (z_ref[0] + z_ref[1]) / jnp.maximum(cnt, 1.0) + r_ref[...]


def _row_spec(width):
  return pl.BlockSpec((_RB, width), lambda i: (i, 0))


def _pad_spec(width):
  return pl.BlockSpec((NC, _RB, width), lambda i: (0, i, 0))


def _full_spec(rows, cols):
  return pl.BlockSpec((rows, cols), lambda i: (0, 0))


_dense_shape = jax.ShapeDtypeStruct((N, D), jnp.float32)

_tc_pre = pl.pallas_call(
    _tc_pre_body,
    grid=(_GRID,),
    in_specs=[_row_spec(D), _full_spec(D, D), _full_spec(D, D),
              _full_spec(1, D)],
    out_specs=[_row_spec(D), _row_spec(D)],
    out_shape=[_dense_shape, _dense_shape],
)

_tc_mid = pl.pallas_call(
    _tc_mid_body,
    grid=(_GRID,),
    in_specs=[_pad_spec(D), _pad_spec(CNTW),
              _row_spec(D), _full_spec(D, D), _full_spec(D, D),
              _full_spec(1, D)],
    out_specs=[_row_spec(D), _row_spec(D)],
    out_shape=[_dense_shape, _dense_shape],
)

_tc_post = pl.pallas_call(
    _tc_post_body,
    grid=(_GRID,),
    in_specs=[_pad_spec(D), _pad_spec(CNTW), _row_spec(D)],
    out_specs=_row_spec(D),
    out_shape=_dense_shape,
)


@jax.jit
def kernel(x, edge_index, W1l, W1r, b1, W2l, W2r, b2):
  npadw = EPP - EPW
  wids = jnp.arange(NW, dtype=jnp.int32)[:, None]
  ks = jnp.arange(npadw, dtype=jnp.int32)[None, :]
  pad_src = (wids * 97 + ks * 131) % N
  pad_dst = N + (wids * 13 + ks) % (NPAD - N)
  src = jnp.concatenate([edge_index[0].reshape(NW, EPW), pad_src], axis=1)
  dst = jnp.concatenate([edge_index[1].reshape(NW, EPW), pad_dst],
                        axis=1).reshape(NW, NCHUNK, CH)
  dstc = edge_index[1].reshape(NW, NCHUNKC, CHC)
  b1r = b1.reshape(1, D)
  b2r = b2.reshape(1, D)
  zrow = jnp.zeros((RPAD, D), jnp.float32)
  zcnt = jnp.zeros((RPAD, CNTW), jnp.float32)
  ones = jnp.ones((CHC, CNTW), jnp.float32)

  (cnt,) = _sc_count_kernel()(dstc, zcnt, ones)
  y1, r1 = _tc_pre(x, W1l, W1r, b1r)
  (z1,) = _sc_segsum_kernel()(y1, src, dst, zrow)
  y2, r2 = _tc_mid(z1, cnt, r1, W2l, W2r, b2r)
  (z2,) = _sc_segsum_kernel()(y2, src, dst, zrow)
  out = _tc_post(z2, cnt, r2)
  return out
